# Initial kernel scaffold; baseline (speedup 1.0000x reference)
import jax
import jax.numpy as jnp
from jax.experimental import pallas as pl


def kernel(x, edge_index, graph_ids, W1, b1, W2, b2, Wf1, bf1, Wf2, bf2):
    raise NotImplementedError("write your pallas kernel here")


# speedup vs baseline: 15.4245x; 15.4245x over previous
import functools

import jax
import jax.numpy as jnp
from jax import lax
from jax.experimental import pallas as pl
from jax.experimental.pallas import tpu as pltpu
from jax.experimental.pallas import tpu_sc as plsc

N = 10000
E = 320000
NG = 64
D = 8

NC, NS = 2, 16
NW = NC * NS

B = 128
KB = 16

NP = 10240
EPW = 10240
EP = EPW * NW
FCH = EPW // (KB * B)
DPW = 2 * EP // NW
DCH = DPW // (KB * B)
RPT_F = NP // NS
RPT_D = 2 * NP // NS

_mesh = plsc.VectorSubcoreMesh(
    core_axis_name="c", subcore_axis_name="s", num_cores=NC, num_subcores=NS)
_sc_params = pltpu.CompilerParams(use_tc_tiling_on_sc=False)


def _deg_body(idx_hbm, ones_hbm, zer_hbm, out_hbm, idx_v, ones_v, acc,
              isem, ssem):
  c = lax.axis_index("c")
  s = lax.axis_index("s")
  wid = s * NC + c
  pltpu.sync_copy(zer_hbm, acc.at[pl.ds(s * RPT_D, RPT_D)])
  pltpu.sync_copy(ones_hbm, ones_v)
  plsc.subcore_barrier()
  pltpu.sync_copy(idx_hbm.at[wid, 0], idx_v.at[0])
  prev = []
  for ci in range(DCH):
    b = ci % 3
    nxt = None
    if ci + 1 < DCH:
      nxt = pltpu.async_copy(idx_hbm.at[wid, ci + 1], idx_v.at[(ci + 1) % 3],
                             isem)
    sds = []
    for j in range(KB):
      sds.append(pltpu.async_copy(ones_v, acc.at[idx_v.at[b, j]], ssem,
                                  add=True))
    for d_ in prev:
      d_.wait()
    prev = sds
    if nxt is not None:
      nxt.wait()
  for d_ in prev:
    d_.wait()
  plsc.subcore_barrier()
  pltpu.sync_copy(acc.at[pl.ds(s * RPT_D, RPT_D)],
                  out_hbm.at[c, pl.ds(s * RPT_D, RPT_D)])


_deg_kernel = functools.partial(
    pl.kernel,
    out_type=jax.ShapeDtypeStruct((NC, 2 * NP, D), jnp.float32),
    mesh=_mesh,
    scratch_types=[
        pltpu.VMEM((3, KB, B), jnp.int32),
        pltpu.VMEM((B, D), jnp.float32),
        pltpu.VMEM_SHARED((2 * NP, D), jnp.float32),
        pltpu.SemaphoreType.DMA,
        pltpu.SemaphoreType.DMA,
    ],
    compiler_params=_sc_params,
)(_deg_body)


def _agg_body(tbl_hbm, sidx_hbm, didx_hbm, zer_hbm, out_hbm,
              sidx, didx, msg, acc, isem, gsem, ssem):
  c = lax.axis_index("c")
  s = lax.axis_index("s")
  wid = s * NC + c
  pltpu.sync_copy(zer_hbm, acc.at[pl.ds(s * RPT_F, RPT_F)])
  plsc.subcore_barrier()
  pltpu.sync_copy(sidx_hbm.at[wid, 0], sidx.at[0])
  pltpu.sync_copy(didx_hbm.at[wid, 0], didx.at[0])
  prev = []
  for ci in range(FCH):
    b = ci % 3
    mb = ci % 2
    nxt = []
    if ci + 1 < FCH:
      nxt.append(pltpu.async_copy(sidx_hbm.at[wid, ci + 1],
                                  sidx.at[(ci + 1) % 3], isem))
      nxt.append(pltpu.async_copy(didx_hbm.at[wid, ci + 1],
                                  didx.at[(ci + 1) % 3], isem))
    gds = []
    for j in range(KB):
      gds.append(pltpu.async_copy(tbl_hbm.at[sidx.at[b, j]], msg.at[mb, j],
                                  gsem.at[j]))
    for d_ in prev:
      d_.wait()
    sds = []
    for j in range(KB):
      gds[j].wait()
      sds.append(pltpu.async_copy(msg.at[mb, j], acc.at[didx.at[b, j]], ssem,
                                  add=True))
    prev = sds
    for d_ in nxt:
      d_.wait()
  for d_ in prev:
    d_.wait()
  plsc.subcore_barrier()
  pltpu.sync_copy(acc.at[pl.ds(s * RPT_F, RPT_F)],
                  out_hbm.at[c, pl.ds(s * RPT_F, RPT_F)])


_agg_kernel = functools.partial(
    pl.kernel,
    out_type=jax.ShapeDtypeStruct((NC, NP, D), jnp.float32),
    mesh=_mesh,
    scratch_types=[
        pltpu.VMEM((3, KB, B), jnp.int32),
        pltpu.VMEM((3, KB, B), jnp.int32),
        pltpu.VMEM((2, KB, B, D), jnp.float32),
        pltpu.VMEM_SHARED((NP, D), jnp.float32),
        pltpu.SemaphoreType.DMA,
        pltpu.SemaphoreType.DMA((KB,)),
        pltpu.SemaphoreType.DMA,
    ],
    compiler_params=_sc_params,
)(_agg_body)


def _norms(dp_ref):
  deg = (dp_ref[0].astype(jnp.float32)
         + dp_ref[1].astype(jnp.float32))
  deg_out = deg[:NP]
  deg_in = deg[NP:]
  ns = jnp.where(deg_out > 0, lax.rsqrt(deg_out), 0.0)
  nd = jnp.where(deg_in > 0, lax.rsqrt(deg_in), 0.0)
  return ns, nd


def _dense1_body(x_ref, w1_ref, dp_ref, hs_ref):
  ns, _ = _norms(dp_ref)
  h = jnp.dot(x_ref[...], w1_ref[...], preferred_element_type=jnp.float32)
  hs_ref[...] = h * ns


def _dense2_body(p_ref, dp_ref, b1_ref, hs2_ref):
  ns, nd = _norms(dp_ref)
  agg = p_ref[0] + p_ref[1]
  out1 = jnp.maximum(agg * nd + b1_ref[...], 0.0)
  hs2_ref[...] = out1 * ns


def _head_body(q_ref, dp_ref, ids_ref, w2_ref, b2_ref, wf1_ref, bf1_ref,
               wf2_ref, bf2_ref, out_ref):
  _, nd = _norms(dp_ref)
  agg = q_ref[0] + q_ref[1]
  h2 = jnp.dot(agg, w2_ref[...], preferred_element_type=jnp.float32)
  h2 = jnp.maximum(h2 * nd + b2_ref[...], 0.0)
  z = (ids_ref[...] == jax.lax.broadcasted_iota(jnp.int32, (1, NG), 1)
       ).astype(jnp.float32)
  sums = lax.dot_general(z, h2, (((0,), (0,)), ((), ())),
                         preferred_element_type=jnp.float32)
  cnts = lax.dot_general(z, jnp.ones((NP, 1), jnp.float32),
                         (((0,), (0,)), ((), ())),
                         preferred_element_type=jnp.float32)
  pooled = sums / jnp.maximum(cnts, 1.0)
  h = jnp.dot(pooled, wf1_ref[...],
              preferred_element_type=jnp.float32) + bf1_ref[...]
  out_ref[...] = jnp.dot(h, wf2_ref[...],
                         preferred_element_type=jnp.float32) + bf2_ref[...]


def _tc_call(body, out_shape):
  return pl.pallas_call(body, out_shape=out_shape)


def kernel(x, edge_index, graph_ids, W1, b1, W2, b2, Wf1, bf1, Wf2, bf2):
  src = edge_index[0].astype(jnp.int32)
  dst = edge_index[1].astype(jnp.int32)
  pad = jnp.full((EP - E,), N, jnp.int32)
  srcp = jnp.concatenate([src, pad])
  dstp = jnp.concatenate([dst, pad])
  src4 = srcp.reshape(NW, FCH, KB, B)
  dst4 = dstp.reshape(NW, FCH, KB, B)
  degidx = jnp.concatenate([srcp, dstp + NP]).reshape(NW, DCH, KB, B)

  ones_b = jnp.ones((B, D), jnp.float32)
  zer_deg = jnp.zeros((RPT_D, D), jnp.float32)
  zer_feat = jnp.zeros((RPT_F, D), jnp.float32)

  dp = _deg_kernel(degidx, ones_b, zer_deg)
  dp3 = dp[:, :, 0:1]

  xp = jnp.pad(x, ((0, NP - N), (0, 0)))
  hs = _tc_call(_dense1_body,
                jax.ShapeDtypeStruct((NP, D), jnp.float32))(xp, W1, dp3)

  p = _agg_kernel(hs, src4, dst4, zer_feat)
  hs2 = _tc_call(_dense2_body,
                 jax.ShapeDtypeStruct((NP, D), jnp.float32))(p, dp3, b1)
  q = _agg_kernel(hs2, src4, dst4, zer_feat)

  idsp = jnp.concatenate(
      [graph_ids.astype(jnp.int32),
       jnp.full((NP - N,), NG, jnp.int32)]).reshape(NP, 1)
  out = _tc_call(_head_body, jax.ShapeDtypeStruct((NG, 10), jnp.float32))(
      q, dp3, idsp, W2, b2, Wf1, bf1, Wf2, bf2)
  return out


# speedup vs baseline: 15.5149x; 1.0059x over previous
import functools

import jax
import jax.numpy as jnp
from jax import lax
from jax.experimental import pallas as pl
from jax.experimental.pallas import tpu as pltpu
from jax.experimental.pallas import tpu_sc as plsc

N = 10000
E = 320000
NG = 64
D = 8

NC, NS = 2, 16
NW = NC * NS

B = 128
KB = 16

NP = 10240
EPW = 10240
EP = EPW * NW
FCH = EPW // (KB * B)
DPW = 2 * EP // NW
DCH = DPW // (KB * B)
RPT_F = NP // NS
RPT_D = 2 * NP // NS
KBB = KB * B

_mesh = plsc.VectorSubcoreMesh(
    core_axis_name="c", subcore_axis_name="s", num_cores=NC, num_subcores=NS)
_sc_params = pltpu.CompilerParams(use_tc_tiling_on_sc=False)


def _deg_body(idx_hbm, ones_hbm, zer_hbm, out_hbm, idx_v, ones_v, acc,
              isem, ssem):
  c = lax.axis_index("c")
  s = lax.axis_index("s")
  wid = s * NC + c
  pltpu.sync_copy(zer_hbm, acc.at[pl.ds(s * RPT_D, RPT_D)])
  pltpu.sync_copy(ones_hbm, ones_v)
  plsc.subcore_barrier()
  pltpu.sync_copy(idx_hbm.at[wid, 0], idx_v.at[0])
  prev = []
  for ci in range(DCH):
    b = ci % 3
    nxt = None
    if ci + 1 < DCH:
      nxt = pltpu.async_copy(idx_hbm.at[wid, ci + 1], idx_v.at[(ci + 1) % 3],
                             isem)
    sd = pltpu.async_copy(ones_v, acc.at[idx_v.at[b]], ssem, add=True)
    for d_ in prev:
      d_.wait()
    prev = [sd]
    if nxt is not None:
      nxt.wait()
  for d_ in prev:
    d_.wait()
  plsc.subcore_barrier()
  pltpu.sync_copy(acc.at[pl.ds(s * RPT_D, RPT_D)],
                  out_hbm.at[c, pl.ds(s * RPT_D, RPT_D)])


_deg_kernel = functools.partial(
    pl.kernel,
    out_type=jax.ShapeDtypeStruct((NC, 2 * NP, D), jnp.float32),
    mesh=_mesh,
    scratch_types=[
        pltpu.VMEM((3, KBB), jnp.int32),
        pltpu.VMEM((KBB, D), jnp.float32),
        pltpu.VMEM_SHARED((2 * NP, D), jnp.float32),
        pltpu.SemaphoreType.DMA,
        pltpu.SemaphoreType.DMA,
    ],
    compiler_params=_sc_params,
)(_deg_body)


def _agg_body(tbl_hbm, sidx_hbm, didx_hbm, zer_hbm, out_hbm,
              sidx, didx, msg, acc, isem, gsem, ssem):
  c = lax.axis_index("c")
  s = lax.axis_index("s")
  wid = s * NC + c
  pltpu.sync_copy(zer_hbm, acc.at[pl.ds(s * RPT_F, RPT_F)])
  plsc.subcore_barrier()
  pltpu.sync_copy(sidx_hbm.at[wid, 0], sidx.at[0])
  pltpu.sync_copy(didx_hbm.at[wid, 0], didx.at[0])
  prev = []
  for ci in range(FCH):
    b = ci % 3
    mb = ci % 2
    nxt = []
    if ci + 1 < FCH:
      nxt.append(pltpu.async_copy(sidx_hbm.at[wid, ci + 1],
                                  sidx.at[(ci + 1) % 3], isem))
      nxt.append(pltpu.async_copy(didx_hbm.at[wid, ci + 1],
                                  didx.at[(ci + 1) % 3], isem))
    g = pltpu.async_copy(tbl_hbm.at[sidx.at[b]], msg.at[mb], gsem)
    for d_ in prev:
      d_.wait()
    g.wait()
    prev = [pltpu.async_copy(msg.at[mb], acc.at[didx.at[b]], ssem, add=True)]
    for d_ in nxt:
      d_.wait()
  for d_ in prev:
    d_.wait()
  plsc.subcore_barrier()
  pltpu.sync_copy(acc.at[pl.ds(s * RPT_F, RPT_F)],
                  out_hbm.at[c, pl.ds(s * RPT_F, RPT_F)])


_agg_kernel = functools.partial(
    pl.kernel,
    out_type=jax.ShapeDtypeStruct((NC, NP, D), jnp.float32),
    mesh=_mesh,
    scratch_types=[
        pltpu.VMEM((3, KBB), jnp.int32),
        pltpu.VMEM((3, KBB), jnp.int32),
        pltpu.VMEM((2, KBB, D), jnp.float32),
        pltpu.VMEM_SHARED((NP, D), jnp.float32),
        pltpu.SemaphoreType.DMA,
        pltpu.SemaphoreType.DMA,
        pltpu.SemaphoreType.DMA,
    ],
    compiler_params=_sc_params,
)(_agg_body)


def _norms(dp_ref):
  deg = (dp_ref[0].astype(jnp.float32)
         + dp_ref[1].astype(jnp.float32))
  deg_out = deg[:NP]
  deg_in = deg[NP:]
  ns = jnp.where(deg_out > 0, lax.rsqrt(deg_out), 0.0)
  nd = jnp.where(deg_in > 0, lax.rsqrt(deg_in), 0.0)
  return ns, nd


def _dense1_body(x_ref, w1_ref, dp_ref, hs_ref):
  ns, _ = _norms(dp_ref)
  h = jnp.dot(x_ref[...], w1_ref[...], preferred_element_type=jnp.float32)
  hs_ref[...] = h * ns


def _dense2_body(p_ref, dp_ref, b1_ref, hs2_ref):
  ns, nd = _norms(dp_ref)
  agg = p_ref[0] + p_ref[1]
  out1 = jnp.maximum(agg * nd + b1_ref[...], 0.0)
  hs2_ref[...] = out1 * ns


def _head_body(q_ref, dp_ref, ids_ref, w2_ref, b2_ref, wf1_ref, bf1_ref,
               wf2_ref, bf2_ref, out_ref):
  _, nd = _norms(dp_ref)
  agg = q_ref[0] + q_ref[1]
  h2 = jnp.dot(agg, w2_ref[...], preferred_element_type=jnp.float32)
  h2 = jnp.maximum(h2 * nd + b2_ref[...], 0.0)
  z = (ids_ref[...] == jax.lax.broadcasted_iota(jnp.int32, (1, NG), 1)
       ).astype(jnp.float32)
  sums = lax.dot_general(z, h2, (((0,), (0,)), ((), ())),
                         preferred_element_type=jnp.float32)
  cnts = lax.dot_general(z, jnp.ones((NP, 1), jnp.float32),
                         (((0,), (0,)), ((), ())),
                         preferred_element_type=jnp.float32)
  pooled = sums / jnp.maximum(cnts, 1.0)
  h = jnp.dot(pooled, wf1_ref[...],
              preferred_element_type=jnp.float32) + bf1_ref[...]
  out_ref[...] = jnp.dot(h, wf2_ref[...],
                         preferred_element_type=jnp.float32) + bf2_ref[...]


def _tc_call(body, out_shape):
  return pl.pallas_call(body, out_shape=out_shape)


def kernel(x, edge_index, graph_ids, W1, b1, W2, b2, Wf1, bf1, Wf2, bf2):
  src = edge_index[0].astype(jnp.int32)
  dst = edge_index[1].astype(jnp.int32)
  pad = jnp.full((EP - E,), N, jnp.int32)
  srcp = jnp.concatenate([src, pad])
  dstp = jnp.concatenate([dst, pad])
  src4 = srcp.reshape(NW, FCH, KBB)
  dst4 = dstp.reshape(NW, FCH, KBB)
  degidx = jnp.concatenate([srcp, dstp + NP]).reshape(NW, DCH, KBB)

  ones_b = jnp.ones((KBB, D), jnp.float32)
  zer_deg = jnp.zeros((RPT_D, D), jnp.float32)
  zer_feat = jnp.zeros((RPT_F, D), jnp.float32)

  dp = _deg_kernel(degidx, ones_b, zer_deg)
  dp3 = dp[:, :, 0:1]

  xp = jnp.pad(x, ((0, NP - N), (0, 0)))
  hs = _tc_call(_dense1_body,
                jax.ShapeDtypeStruct((NP, D), jnp.float32))(xp, W1, dp3)

  p = _agg_kernel(hs, src4, dst4, zer_feat)
  hs2 = _tc_call(_dense2_body,
                 jax.ShapeDtypeStruct((NP, D), jnp.float32))(p, dp3, b1)
  q = _agg_kernel(hs2, src4, dst4, zer_feat)

  idsp = jnp.concatenate(
      [graph_ids.astype(jnp.int32),
       jnp.full((NP - N,), NG, jnp.int32)]).reshape(NP, 1)
  out = _tc_call(_head_body, jax.ShapeDtypeStruct((NG, 10), jnp.float32))(
      q, dp3, idsp, W2, b2, Wf1, bf1, Wf2, bf2)
  return out


# speedup vs baseline: 23.1635x; 1.4930x over previous
import functools

import jax
import jax.numpy as jnp
from jax import lax
from jax.experimental import pallas as pl
from jax.experimental.pallas import tpu as pltpu
from jax.experimental.pallas import tpu_sc as plsc

N = 10000
E = 320000
NG = 64
D = 8

NC, NS = 2, 16
NW = NC * NS
EW = E // NW

KBB = 2048
CHS = [KBB, KBB, KBB, KBB, EW - 4 * KBB]
NCH = len(CHS)

NP = 10240
RPT = NP // NS

_mesh = plsc.VectorSubcoreMesh(
    core_axis_name="c", subcore_axis_name="s", num_cores=NC, num_subcores=NS)
_sc_params = pltpu.CompilerParams(use_tc_tiling_on_sc=False)


def _deg_body(ei_hbm, ones_hbm, zer_hbm, out_hbm, idx_v, ones_v,
              acc_o, acc_i, isem, ssem):
  c = lax.axis_index("c")
  s = lax.axis_index("s")
  wid = s * NC + c
  base = pl.multiple_of(wid * EW, 8)
  pltpu.sync_copy(zer_hbm, acc_o.at[pl.ds(s * RPT, RPT)])
  pltpu.sync_copy(zer_hbm, acc_i.at[pl.ds(s * RPT, RPT)])
  pltpu.sync_copy(ones_hbm, ones_v)
  plsc.subcore_barrier()
  pltpu.sync_copy(ei_hbm.at[0, pl.ds(base, CHS[0])],
                  idx_v.at[0, 0, pl.ds(0, CHS[0])])
  pltpu.sync_copy(ei_hbm.at[1, pl.ds(base, CHS[0])],
                  idx_v.at[0, 1, pl.ds(0, CHS[0])])
  prev = []
  for ci in range(NCH):
    b = ci % 3
    n = CHS[ci]
    nxt = []
    if ci + 1 < NCH:
      n1 = CHS[ci + 1]
      off = ci * KBB + KBB
      nb = (ci + 1) % 3
      nxt.append(pltpu.async_copy(ei_hbm.at[0, pl.ds(base + off, n1)],
                                  idx_v.at[nb, 0, pl.ds(0, n1)], isem))
      nxt.append(pltpu.async_copy(ei_hbm.at[1, pl.ds(base + off, n1)],
                                  idx_v.at[nb, 1, pl.ds(0, n1)], isem))
    sds = [
        pltpu.async_copy(ones_v.at[pl.ds(0, n)],
                         acc_o.at[idx_v.at[b, 0, pl.ds(0, n)]], ssem,
                         add=True),
        pltpu.async_copy(ones_v.at[pl.ds(0, n)],
                         acc_i.at[idx_v.at[b, 1, pl.ds(0, n)]], ssem,
                         add=True),
    ]
    for d_ in prev:
      d_.wait()
    prev = sds
    for d_ in nxt:
      d_.wait()
  for d_ in prev:
    d_.wait()
  plsc.subcore_barrier()
  pltpu.sync_copy(acc_o.at[pl.ds(s * RPT, RPT)],
                  out_hbm.at[c, 0, pl.ds(s * RPT, RPT)])
  pltpu.sync_copy(acc_i.at[pl.ds(s * RPT, RPT)],
                  out_hbm.at[c, 1, pl.ds(s * RPT, RPT)])


_deg_kernel = functools.partial(
    pl.kernel,
    out_type=jax.ShapeDtypeStruct((NC, 2, NP, D), jnp.float32),
    mesh=_mesh,
    scratch_types=[
        pltpu.VMEM((3, 2, KBB), jnp.int32),
        pltpu.VMEM((KBB, D), jnp.float32),
        pltpu.VMEM_SHARED((NP, D), jnp.float32),
        pltpu.VMEM_SHARED((NP, D), jnp.float32),
        pltpu.SemaphoreType.DMA,
        pltpu.SemaphoreType.DMA,
    ],
    compiler_params=_sc_params,
)(_deg_body)


def _agg_body(tbl_hbm, ei_hbm, zer_hbm, out_hbm,
              idx_v, msg, acc, isem, gsem, ssem):
  c = lax.axis_index("c")
  s = lax.axis_index("s")
  wid = s * NC + c
  base = pl.multiple_of(wid * EW, 8)
  pltpu.sync_copy(zer_hbm, acc.at[pl.ds(s * RPT, RPT)])
  plsc.subcore_barrier()
  pltpu.sync_copy(ei_hbm.at[0, pl.ds(base, CHS[0])],
                  idx_v.at[0, 0, pl.ds(0, CHS[0])])
  pltpu.sync_copy(ei_hbm.at[1, pl.ds(base, CHS[0])],
                  idx_v.at[0, 1, pl.ds(0, CHS[0])])
  prev = []
  for ci in range(NCH):
    b = ci % 3
    mb = ci % 2
    n = CHS[ci]
    nxt = []
    if ci + 1 < NCH:
      n1 = CHS[ci + 1]
      off = ci * KBB + KBB
      nb = (ci + 1) % 3
      nxt.append(pltpu.async_copy(ei_hbm.at[0, pl.ds(base + off, n1)],
                                  idx_v.at[nb, 0, pl.ds(0, n1)], isem))
      nxt.append(pltpu.async_copy(ei_hbm.at[1, pl.ds(base + off, n1)],
                                  idx_v.at[nb, 1, pl.ds(0, n1)], isem))
    g = pltpu.async_copy(tbl_hbm.at[idx_v.at[b, 0, pl.ds(0, n)]],
                         msg.at[mb, pl.ds(0, n)], gsem)
    for d_ in prev:
      d_.wait()
    g.wait()
    prev = [pltpu.async_copy(msg.at[mb, pl.ds(0, n)],
                             acc.at[idx_v.at[b, 1, pl.ds(0, n)]], ssem,
                             add=True)]
    for d_ in nxt:
      d_.wait()
  for d_ in prev:
    d_.wait()
  plsc.subcore_barrier()
  pltpu.sync_copy(acc.at[pl.ds(s * RPT, RPT)],
                  out_hbm.at[c, pl.ds(s * RPT, RPT)])


_agg_kernel = functools.partial(
    pl.kernel,
    out_type=jax.ShapeDtypeStruct((NC, NP, D), jnp.float32),
    mesh=_mesh,
    scratch_types=[
        pltpu.VMEM((3, 2, KBB), jnp.int32),
        pltpu.VMEM((2, KBB, D), jnp.float32),
        pltpu.VMEM_SHARED((NP, D), jnp.float32),
        pltpu.SemaphoreType.DMA,
        pltpu.SemaphoreType.DMA,
        pltpu.SemaphoreType.DMA,
    ],
    compiler_params=_sc_params,
)(_agg_body)


def _norms(dp_ref):
  deg_out = (dp_ref[0, 0].astype(jnp.float32)
             + dp_ref[1, 0].astype(jnp.float32))[:N]
  deg_in = (dp_ref[0, 1].astype(jnp.float32)
            + dp_ref[1, 1].astype(jnp.float32))[:N]
  ns = jnp.where(deg_out > 0, lax.rsqrt(deg_out), 0.0)
  nd = jnp.where(deg_in > 0, lax.rsqrt(deg_in), 0.0)
  return ns, nd


def _dense1_body(x_ref, w1_ref, dp_ref, hs_ref):
  ns, _ = _norms(dp_ref)
  h = jnp.dot(x_ref[...], w1_ref[...], preferred_element_type=jnp.float32)
  hs_ref[...] = h * ns


def _dense2_body(p_ref, dp_ref, b1_ref, hs2_ref):
  ns, nd = _norms(dp_ref)
  agg = p_ref[0, :N] + p_ref[1, :N]
  out1 = jnp.maximum(agg * nd + b1_ref[...], 0.0)
  hs2_ref[...] = out1 * ns


def _head_body(q_ref, dp_ref, ids_ref, w2_ref, b2_ref, wf1_ref, bf1_ref,
               wf2_ref, bf2_ref, out_ref):
  _, nd = _norms(dp_ref)
  agg = q_ref[0, :N] + q_ref[1, :N]
  h2 = jnp.dot(agg, w2_ref[...], preferred_element_type=jnp.float32)
  h2 = jnp.maximum(h2 * nd + b2_ref[...], 0.0)
  z = (ids_ref[...] == jax.lax.broadcasted_iota(jnp.int32, (1, NG), 1)
       ).astype(jnp.float32)
  sums = lax.dot_general(z, h2, (((0,), (0,)), ((), ())),
                         preferred_element_type=jnp.float32)
  cnts = lax.dot_general(z, jnp.ones((N, 1), jnp.float32),
                         (((0,), (0,)), ((), ())),
                         preferred_element_type=jnp.float32)
  pooled = sums / jnp.maximum(cnts, 1.0)
  h = jnp.dot(pooled, wf1_ref[...],
              preferred_element_type=jnp.float32) + bf1_ref[...]
  out_ref[...] = jnp.dot(h, wf2_ref[...],
                         preferred_element_type=jnp.float32) + bf2_ref[...]


def _tc_call(body, out_shape):
  return pl.pallas_call(body, out_shape=out_shape)


def kernel(x, edge_index, graph_ids, W1, b1, W2, b2, Wf1, bf1, Wf2, bf2):
  ei = edge_index.astype(jnp.int32)

  ones_b = jnp.ones((KBB, D), jnp.float32)
  zer = jnp.zeros((RPT, D), jnp.float32)

  dp = _deg_kernel(ei, ones_b, zer)
  dp4 = dp[:, :, :, 0:1]

  hs = _tc_call(_dense1_body,
                jax.ShapeDtypeStruct((N, D), jnp.float32))(x, W1, dp4)

  p = _agg_kernel(hs, ei, zer)
  hs2 = _tc_call(_dense2_body,
                 jax.ShapeDtypeStruct((N, D), jnp.float32))(p, dp4, b1)
  q = _agg_kernel(hs2, ei, zer)

  idsp = graph_ids.astype(jnp.int32).reshape(N, 1)
  out = _tc_call(_head_body, jax.ShapeDtypeStruct((NG, 10), jnp.float32))(
      q, dp4, idsp, W2, b2, Wf1, bf1, Wf2, bf2)
  return out


# speedup vs baseline: 23.6327x; 1.0203x over previous
import functools

import jax
import jax.numpy as jnp
from jax import lax
from jax.experimental import pallas as pl
from jax.experimental.pallas import tpu as pltpu
from jax.experimental.pallas import tpu_sc as plsc

N = 10000
E = 320000
NG = 64
D = 8

NC, NS = 2, 16
NW = NC * NS
EW = E // NW

KBB = 2048
CHS = [KBB, KBB, KBB, KBB, EW - 4 * KBB]
NCH = len(CHS)

NP = 10240
RPT = NP // NS

_mesh = plsc.VectorSubcoreMesh(
    core_axis_name="c", subcore_axis_name="s", num_cores=NC, num_subcores=NS)
_sc_params = pltpu.CompilerParams(use_tc_tiling_on_sc=False)


def _deg_body(ei_hbm, ones_hbm, zer_hbm, out_hbm, idx_v, ones_v,
              acc_o, acc_i, isem, ssem):
  c = lax.axis_index("c")
  s = lax.axis_index("s")
  wid = s * NC + c
  base = pl.multiple_of(wid * EW, 8)
  pro = [
      pltpu.async_copy(zer_hbm, acc_o.at[pl.ds(s * RPT, RPT)], isem),
      pltpu.async_copy(zer_hbm, acc_i.at[pl.ds(s * RPT, RPT)], isem),
      pltpu.async_copy(ones_hbm, ones_v, isem),
      pltpu.async_copy(ei_hbm.at[0, pl.ds(base, CHS[0])],
                       idx_v.at[0, 0, pl.ds(0, CHS[0])], isem),
      pltpu.async_copy(ei_hbm.at[1, pl.ds(base, CHS[0])],
                       idx_v.at[0, 1, pl.ds(0, CHS[0])], isem),
  ]
  for d_ in pro:
    d_.wait()
  plsc.subcore_barrier()
  prev = []
  for ci in range(NCH):
    b = ci % 3
    n = CHS[ci]
    nxt = []
    if ci + 1 < NCH:
      n1 = CHS[ci + 1]
      off = ci * KBB + KBB
      nb = (ci + 1) % 3
      nxt.append(pltpu.async_copy(ei_hbm.at[0, pl.ds(base + off, n1)],
                                  idx_v.at[nb, 0, pl.ds(0, n1)], isem))
      nxt.append(pltpu.async_copy(ei_hbm.at[1, pl.ds(base + off, n1)],
                                  idx_v.at[nb, 1, pl.ds(0, n1)], isem))
    sds = [
        pltpu.async_copy(ones_v.at[pl.ds(0, n)],
                         acc_o.at[idx_v.at[b, 0, pl.ds(0, n)]], ssem,
                         add=True),
        pltpu.async_copy(ones_v.at[pl.ds(0, n)],
                         acc_i.at[idx_v.at[b, 1, pl.ds(0, n)]], ssem,
                         add=True),
    ]
    for d_ in prev:
      d_.wait()
    prev = sds
    for d_ in nxt:
      d_.wait()
  for d_ in prev:
    d_.wait()
  plsc.subcore_barrier()
  pltpu.sync_copy(acc_o.at[pl.ds(s * RPT, RPT)],
                  out_hbm.at[c, 0, pl.ds(s * RPT, RPT)])
  pltpu.sync_copy(acc_i.at[pl.ds(s * RPT, RPT)],
                  out_hbm.at[c, 1, pl.ds(s * RPT, RPT)])


_deg_kernel = functools.partial(
    pl.kernel,
    out_type=jax.ShapeDtypeStruct((NC, 2, NP, D), jnp.float32),
    mesh=_mesh,
    scratch_types=[
        pltpu.VMEM((3, 2, KBB), jnp.int32),
        pltpu.VMEM((KBB, D), jnp.float32),
        pltpu.VMEM_SHARED((NP, D), jnp.float32),
        pltpu.VMEM_SHARED((NP, D), jnp.float32),
        pltpu.SemaphoreType.DMA,
        pltpu.SemaphoreType.DMA,
    ],
    compiler_params=_sc_params,
)(_deg_body)


def _agg_body(tbl_hbm, ei_hbm, zer_hbm, out_hbm,
              idx_v, msg, acc, isem, gsem, ssem):
  c = lax.axis_index("c")
  s = lax.axis_index("s")
  wid = s * NC + c
  base = pl.multiple_of(wid * EW, 8)
  pro = [
      pltpu.async_copy(zer_hbm, acc.at[pl.ds(s * RPT, RPT)], isem),
      pltpu.async_copy(ei_hbm.at[0, pl.ds(base, CHS[0])],
                       idx_v.at[0, 0, pl.ds(0, CHS[0])], isem),
      pltpu.async_copy(ei_hbm.at[1, pl.ds(base, CHS[0])],
                       idx_v.at[0, 1, pl.ds(0, CHS[0])], isem),
  ]
  for d_ in pro:
    d_.wait()
  plsc.subcore_barrier()
  prev = []
  for ci in range(NCH):
    b = ci % 3
    mb = ci % 2
    n = CHS[ci]
    nxt = []
    if ci + 1 < NCH:
      n1 = CHS[ci + 1]
      off = ci * KBB + KBB
      nb = (ci + 1) % 3
      nxt.append(pltpu.async_copy(ei_hbm.at[0, pl.ds(base + off, n1)],
                                  idx_v.at[nb, 0, pl.ds(0, n1)], isem))
      nxt.append(pltpu.async_copy(ei_hbm.at[1, pl.ds(base + off, n1)],
                                  idx_v.at[nb, 1, pl.ds(0, n1)], isem))
    g = pltpu.async_copy(tbl_hbm.at[idx_v.at[b, 0, pl.ds(0, n)]],
                         msg.at[mb, pl.ds(0, n)], gsem)
    for d_ in prev:
      d_.wait()
    g.wait()
    prev = [pltpu.async_copy(msg.at[mb, pl.ds(0, n)],
                             acc.at[idx_v.at[b, 1, pl.ds(0, n)]], ssem,
                             add=True)]
    for d_ in nxt:
      d_.wait()
  for d_ in prev:
    d_.wait()
  plsc.subcore_barrier()
  pltpu.sync_copy(acc.at[pl.ds(s * RPT, RPT)],
                  out_hbm.at[c, pl.ds(s * RPT, RPT)])


_agg_kernel = functools.partial(
    pl.kernel,
    out_type=jax.ShapeDtypeStruct((NC, NP, D), jnp.float32),
    mesh=_mesh,
    scratch_types=[
        pltpu.VMEM((3, 2, KBB), jnp.int32),
        pltpu.VMEM((2, KBB, D), jnp.float32),
        pltpu.VMEM_SHARED((NP, D), jnp.float32),
        pltpu.SemaphoreType.DMA,
        pltpu.SemaphoreType.DMA,
        pltpu.SemaphoreType.DMA,
    ],
    compiler_params=_sc_params,
)(_agg_body)


def _norms(dp_ref):
  deg_out = (dp_ref[0, 0].astype(jnp.float32)
             + dp_ref[1, 0].astype(jnp.float32))[:N]
  deg_in = (dp_ref[0, 1].astype(jnp.float32)
            + dp_ref[1, 1].astype(jnp.float32))[:N]
  ns = jnp.where(deg_out > 0, lax.rsqrt(deg_out), 0.0)
  nd = jnp.where(deg_in > 0, lax.rsqrt(deg_in), 0.0)
  return ns, nd


def _dense1_body(x_ref, w1_ref, dp_ref, hs_ref):
  ns, _ = _norms(dp_ref)
  h = jnp.dot(x_ref[...], w1_ref[...], preferred_element_type=jnp.float32)
  hs_ref[...] = h * ns


def _dense2_body(p_ref, dp_ref, b1_ref, hs2_ref):
  ns, nd = _norms(dp_ref)
  agg = p_ref[0, :N] + p_ref[1, :N]
  out1 = jnp.maximum(agg * nd + b1_ref[...], 0.0)
  hs2_ref[...] = out1 * ns


def _head_body(q_ref, dp_ref, ids_ref, w2_ref, b2_ref, wf1_ref, bf1_ref,
               wf2_ref, bf2_ref, out_ref):
  _, nd = _norms(dp_ref)
  agg = q_ref[0, :N] + q_ref[1, :N]
  h2 = jnp.dot(agg, w2_ref[...], preferred_element_type=jnp.float32)
  h2 = jnp.maximum(h2 * nd + b2_ref[...], 0.0)
  z = (ids_ref[...] == jax.lax.broadcasted_iota(jnp.int32, (1, NG), 1)
       ).astype(jnp.float32)
  sums = lax.dot_general(z, h2, (((0,), (0,)), ((), ())),
                         preferred_element_type=jnp.float32)
  cnts = lax.dot_general(z, jnp.ones((N, 1), jnp.float32),
                         (((0,), (0,)), ((), ())),
                         preferred_element_type=jnp.float32)
  pooled = sums / jnp.maximum(cnts, 1.0)
  h = jnp.dot(pooled, wf1_ref[...],
              preferred_element_type=jnp.float32) + bf1_ref[...]
  out_ref[...] = jnp.dot(h, wf2_ref[...],
                         preferred_element_type=jnp.float32) + bf2_ref[...]


def _tc_call(body, out_shape):
  return pl.pallas_call(body, out_shape=out_shape)


def kernel(x, edge_index, graph_ids, W1, b1, W2, b2, Wf1, bf1, Wf2, bf2):
  ei = edge_index.astype(jnp.int32)

  ones_b = jnp.ones((KBB, D), jnp.float32)
  zer = jnp.zeros((RPT, D), jnp.float32)

  dp = _deg_kernel(ei, ones_b, zer)
  dp4 = dp[:, :, :, 0:1]

  hs = _tc_call(_dense1_body,
                jax.ShapeDtypeStruct((N, D), jnp.float32))(x, W1, dp4)

  p = _agg_kernel(hs, ei, zer)
  hs2 = _tc_call(_dense2_body,
                 jax.ShapeDtypeStruct((N, D), jnp.float32))(p, dp4, b1)
  q = _agg_kernel(hs2, ei, zer)

  idsp = graph_ids.astype(jnp.int32).reshape(N, 1)
  out = _tc_call(_head_body, jax.ShapeDtypeStruct((NG, 10), jnp.float32))(
      q, dp4, idsp, W2, b2, Wf1, bf1, Wf2, bf2)
  return out


# speedup vs baseline: 23.7707x; 1.0058x over previous
import functools

import jax
import jax.numpy as jnp
from jax import lax
from jax.experimental import pallas as pl
from jax.experimental.pallas import tpu as pltpu
from jax.experimental.pallas import tpu_sc as plsc

N = 10000
E = 320000
NG = 64
D = 8

NC, NS = 2, 16
NW = NC * NS
EW = E // NW

KBB = 2560
CHS = [KBB, KBB, KBB, EW - 3 * KBB]
NCH = len(CHS)

NP = 10240
RPT = NP // NS

_mesh = plsc.VectorSubcoreMesh(
    core_axis_name="c", subcore_axis_name="s", num_cores=NC, num_subcores=NS)
_sc_params = pltpu.CompilerParams(use_tc_tiling_on_sc=False)


def _deg_body(ei_hbm, ones_hbm, zer_hbm, out_hbm, idx_v, ones_v,
              acc_o, acc_i, isem, ssem):
  c = lax.axis_index("c")
  s = lax.axis_index("s")
  wid = s * NC + c
  base = pl.multiple_of(wid * EW, 8)
  pro = [
      pltpu.async_copy(zer_hbm, acc_o.at[pl.ds(s * RPT, RPT)], isem),
      pltpu.async_copy(zer_hbm, acc_i.at[pl.ds(s * RPT, RPT)], isem),
      pltpu.async_copy(ones_hbm, ones_v, isem),
      pltpu.async_copy(ei_hbm.at[0, pl.ds(base, CHS[0])],
                       idx_v.at[0, 0, pl.ds(0, CHS[0])], isem),
      pltpu.async_copy(ei_hbm.at[1, pl.ds(base, CHS[0])],
                       idx_v.at[0, 1, pl.ds(0, CHS[0])], isem),
  ]
  for d_ in pro:
    d_.wait()
  plsc.subcore_barrier()
  prev = []
  for ci in range(NCH):
    b = ci % 3
    n = CHS[ci]
    nxt = []
    if ci + 1 < NCH:
      n1 = CHS[ci + 1]
      off = ci * KBB + KBB
      nb = (ci + 1) % 3
      nxt.append(pltpu.async_copy(ei_hbm.at[0, pl.ds(base + off, n1)],
                                  idx_v.at[nb, 0, pl.ds(0, n1)], isem))
      nxt.append(pltpu.async_copy(ei_hbm.at[1, pl.ds(base + off, n1)],
                                  idx_v.at[nb, 1, pl.ds(0, n1)], isem))
    sds = [
        pltpu.async_copy(ones_v.at[pl.ds(0, n)],
                         acc_o.at[idx_v.at[b, 0, pl.ds(0, n)]], ssem,
                         add=True),
        pltpu.async_copy(ones_v.at[pl.ds(0, n)],
                         acc_i.at[idx_v.at[b, 1, pl.ds(0, n)]], ssem,
                         add=True),
    ]
    for d_ in prev:
      d_.wait()
    prev = sds
    for d_ in nxt:
      d_.wait()
  for d_ in prev:
    d_.wait()
  plsc.subcore_barrier()
  pltpu.sync_copy(acc_o.at[pl.ds(s * RPT, RPT)],
                  out_hbm.at[c, 0, pl.ds(s * RPT, RPT)])
  pltpu.sync_copy(acc_i.at[pl.ds(s * RPT, RPT)],
                  out_hbm.at[c, 1, pl.ds(s * RPT, RPT)])


_deg_kernel = functools.partial(
    pl.kernel,
    out_type=jax.ShapeDtypeStruct((NC, 2, NP, D), jnp.float32),
    mesh=_mesh,
    scratch_types=[
        pltpu.VMEM((3, 2, KBB), jnp.int32),
        pltpu.VMEM((KBB, D), jnp.float32),
        pltpu.VMEM_SHARED((NP, D), jnp.float32),
        pltpu.VMEM_SHARED((NP, D), jnp.float32),
        pltpu.SemaphoreType.DMA,
        pltpu.SemaphoreType.DMA,
    ],
    compiler_params=_sc_params,
)(_deg_body)


def _agg_body(tbl_hbm, ei_hbm, zer_hbm, out_hbm,
              idx_v, msg, acc, isem, gsem, ssem):
  c = lax.axis_index("c")
  s = lax.axis_index("s")
  wid = s * NC + c
  base = pl.multiple_of(wid * EW, 8)
  pro = [
      pltpu.async_copy(zer_hbm, acc.at[pl.ds(s * RPT, RPT)], isem),
      pltpu.async_copy(ei_hbm.at[0, pl.ds(base, CHS[0])],
                       idx_v.at[0, 0, pl.ds(0, CHS[0])], isem),
      pltpu.async_copy(ei_hbm.at[1, pl.ds(base, CHS[0])],
                       idx_v.at[0, 1, pl.ds(0, CHS[0])], isem),
  ]
  for d_ in pro:
    d_.wait()
  plsc.subcore_barrier()
  prev = []
  for ci in range(NCH):
    b = ci % 3
    mb = ci % 2
    n = CHS[ci]
    nxt = []
    if ci + 1 < NCH:
      n1 = CHS[ci + 1]
      off = ci * KBB + KBB
      nb = (ci + 1) % 3
      nxt.append(pltpu.async_copy(ei_hbm.at[0, pl.ds(base + off, n1)],
                                  idx_v.at[nb, 0, pl.ds(0, n1)], isem))
      nxt.append(pltpu.async_copy(ei_hbm.at[1, pl.ds(base + off, n1)],
                                  idx_v.at[nb, 1, pl.ds(0, n1)], isem))
    g = pltpu.async_copy(tbl_hbm.at[idx_v.at[b, 0, pl.ds(0, n)]],
                         msg.at[mb, pl.ds(0, n)], gsem)
    for d_ in prev:
      d_.wait()
    g.wait()
    prev = [pltpu.async_copy(msg.at[mb, pl.ds(0, n)],
                             acc.at[idx_v.at[b, 1, pl.ds(0, n)]], ssem,
                             add=True)]
    for d_ in nxt:
      d_.wait()
  for d_ in prev:
    d_.wait()
  plsc.subcore_barrier()
  pltpu.sync_copy(acc.at[pl.ds(s * RPT, RPT)],
                  out_hbm.at[c, pl.ds(s * RPT, RPT)])


_agg_kernel = functools.partial(
    pl.kernel,
    out_type=jax.ShapeDtypeStruct((NC, NP, D), jnp.float32),
    mesh=_mesh,
    scratch_types=[
        pltpu.VMEM((3, 2, KBB), jnp.int32),
        pltpu.VMEM((2, KBB, D), jnp.float32),
        pltpu.VMEM_SHARED((NP, D), jnp.float32),
        pltpu.SemaphoreType.DMA,
        pltpu.SemaphoreType.DMA,
        pltpu.SemaphoreType.DMA,
    ],
    compiler_params=_sc_params,
)(_agg_body)


def _norms(dp_ref):
  deg_out = (dp_ref[0, 0].astype(jnp.float32)
             + dp_ref[1, 0].astype(jnp.float32))[:N]
  deg_in = (dp_ref[0, 1].astype(jnp.float32)
            + dp_ref[1, 1].astype(jnp.float32))[:N]
  ns = jnp.where(deg_out > 0, lax.rsqrt(deg_out), 0.0)
  nd = jnp.where(deg_in > 0, lax.rsqrt(deg_in), 0.0)
  return ns, nd


def _dense1_body(x_ref, w1_ref, dp_ref, hs_ref):
  ns, _ = _norms(dp_ref)
  h = jnp.dot(x_ref[...], w1_ref[...], preferred_element_type=jnp.float32)
  hs_ref[...] = h * ns


def _dense2_body(p_ref, dp_ref, b1_ref, hs2_ref):
  ns, nd = _norms(dp_ref)
  agg = p_ref[0, :N] + p_ref[1, :N]
  out1 = jnp.maximum(agg * nd + b1_ref[...], 0.0)
  hs2_ref[...] = out1 * ns


def _head_body(q_ref, dp_ref, ids_ref, w2_ref, b2_ref, wf1_ref, bf1_ref,
               wf2_ref, bf2_ref, out_ref):
  _, nd = _norms(dp_ref)
  agg = q_ref[0, :N] + q_ref[1, :N]
  h2 = jnp.dot(agg, w2_ref[...], preferred_element_type=jnp.float32)
  h2 = jnp.maximum(h2 * nd + b2_ref[...], 0.0)
  z = (ids_ref[...] == jax.lax.broadcasted_iota(jnp.int32, (1, NG), 1)
       ).astype(jnp.float32)
  sums = lax.dot_general(z, h2, (((0,), (0,)), ((), ())),
                         preferred_element_type=jnp.float32)
  cnts = lax.dot_general(z, jnp.ones((N, 1), jnp.float32),
                         (((0,), (0,)), ((), ())),
                         preferred_element_type=jnp.float32)
  pooled = sums / jnp.maximum(cnts, 1.0)
  h = jnp.dot(pooled, wf1_ref[...],
              preferred_element_type=jnp.float32) + bf1_ref[...]
  out_ref[...] = jnp.dot(h, wf2_ref[...],
                         preferred_element_type=jnp.float32) + bf2_ref[...]


def _tc_call(body, out_shape):
  return pl.pallas_call(body, out_shape=out_shape)


def kernel(x, edge_index, graph_ids, W1, b1, W2, b2, Wf1, bf1, Wf2, bf2):
  ei = edge_index.astype(jnp.int32)

  ones_b = jnp.ones((KBB, D), jnp.float32)
  zer = jnp.zeros((RPT, D), jnp.float32)

  dp = _deg_kernel(ei, ones_b, zer)
  dp4 = dp[:, :, :, 0:1]

  hs = _tc_call(_dense1_body,
                jax.ShapeDtypeStruct((N, D), jnp.float32))(x, W1, dp4)

  p = _agg_kernel(hs, ei, zer)
  hs2 = _tc_call(_dense2_body,
                 jax.ShapeDtypeStruct((N, D), jnp.float32))(p, dp4, b1)
  q = _agg_kernel(hs2, ei, zer)

  idsp = graph_ids.astype(jnp.int32).reshape(N, 1)
  out = _tc_call(_head_body, jax.ShapeDtypeStruct((NG, 10), jnp.float32))(
      q, dp4, idsp, W2, b2, Wf1, bf1, Wf2, bf2)
  return out


# speedup vs baseline: 23.8033x; 1.0014x over previous
import functools

import jax
import jax.numpy as jnp
from jax import lax
from jax.experimental import pallas as pl
from jax.experimental.pallas import tpu as pltpu
from jax.experimental.pallas import tpu_sc as plsc

N = 10000
E = 320000
NG = 64
D = 8

NC, NS = 2, 16
NW = NC * NS
EW = E // NW

KBB = 2560
CHS = [KBB, KBB, KBB, EW - 3 * KBB]
NCH = len(CHS)

NP = 10240
RPT = NP // NS

_mesh = plsc.VectorSubcoreMesh(
    core_axis_name="c", subcore_axis_name="s", num_cores=NC, num_subcores=NS)
_sc_params = pltpu.CompilerParams(use_tc_tiling_on_sc=False)


def _deg_body(ei_hbm, ones_hbm, zer_hbm, out_hbm, idx_v, ones_v,
              acc_o, acc_i, isem, ssem):
  c = lax.axis_index("c")
  s = lax.axis_index("s")
  wid = s * NC + c
  base = pl.multiple_of(wid * EW, 8)
  pro = [
      pltpu.async_copy(zer_hbm, acc_o.at[pl.ds(s * RPT, RPT)], isem),
      pltpu.async_copy(zer_hbm, acc_i.at[pl.ds(s * RPT, RPT)], isem),
      pltpu.async_copy(ones_hbm, ones_v, isem),
      pltpu.async_copy(ei_hbm.at[0, pl.ds(base, CHS[0])],
                       idx_v.at[0, 0, pl.ds(0, CHS[0])], isem),
      pltpu.async_copy(ei_hbm.at[1, pl.ds(base, CHS[0])],
                       idx_v.at[0, 1, pl.ds(0, CHS[0])], isem),
  ]
  for d_ in pro:
    d_.wait()
  plsc.subcore_barrier()
  prev = []
  for ci in range(NCH):
    b = ci % 3
    n = CHS[ci]
    nxt = []
    if ci + 1 < NCH:
      n1 = CHS[ci + 1]
      off = ci * KBB + KBB
      nb = (ci + 1) % 3
      nxt.append(pltpu.async_copy(ei_hbm.at[0, pl.ds(base + off, n1)],
                                  idx_v.at[nb, 0, pl.ds(0, n1)], isem))
      nxt.append(pltpu.async_copy(ei_hbm.at[1, pl.ds(base + off, n1)],
                                  idx_v.at[nb, 1, pl.ds(0, n1)], isem))
    sds = [
        pltpu.async_copy(ones_v.at[pl.ds(0, n)],
                         acc_o.at[idx_v.at[b, 0, pl.ds(0, n)]], ssem,
                         add=True),
        pltpu.async_copy(ones_v.at[pl.ds(0, n)],
                         acc_i.at[idx_v.at[b, 1, pl.ds(0, n)]], ssem,
                         add=True),
    ]
    for d_ in prev:
      d_.wait()
    prev = sds
    for d_ in nxt:
      d_.wait()
  for d_ in prev:
    d_.wait()
  plsc.subcore_barrier()
  pltpu.sync_copy(acc_o.at[pl.ds(s * RPT, RPT)],
                  out_hbm.at[c, 0, pl.ds(s * RPT, RPT)])
  pltpu.sync_copy(acc_i.at[pl.ds(s * RPT, RPT)],
                  out_hbm.at[c, 1, pl.ds(s * RPT, RPT)])


_deg_kernel = functools.partial(
    pl.kernel,
    out_type=jax.ShapeDtypeStruct((NC, 2, NP, D), jnp.float32),
    mesh=_mesh,
    scratch_types=[
        pltpu.VMEM((3, 2, KBB), jnp.int32),
        pltpu.VMEM((KBB, D), jnp.float32),
        pltpu.VMEM_SHARED((NP, D), jnp.float32),
        pltpu.VMEM_SHARED((NP, D), jnp.float32),
        pltpu.SemaphoreType.DMA,
        pltpu.SemaphoreType.DMA,
    ],
    compiler_params=_sc_params,
)(_deg_body)


def _agg_body(tbl_hbm, ei_hbm, zer_hbm, out_hbm,
              idx_v, msg, acc, isem, gsem, ssem):
  c = lax.axis_index("c")
  s = lax.axis_index("s")
  wid = s * NC + c
  base = pl.multiple_of(wid * EW, 8)
  pro = [
      pltpu.async_copy(zer_hbm, acc.at[pl.ds(s * RPT, RPT)], isem),
      pltpu.async_copy(ei_hbm.at[0, pl.ds(base, CHS[0])],
                       idx_v.at[0, 0, pl.ds(0, CHS[0])], isem),
      pltpu.async_copy(ei_hbm.at[1, pl.ds(base, CHS[0])],
                       idx_v.at[0, 1, pl.ds(0, CHS[0])], isem),
  ]
  for d_ in pro:
    d_.wait()
  plsc.subcore_barrier()
  prev = []
  for ci in range(NCH):
    b = ci % 3
    mb = ci % 2
    n = CHS[ci]
    nxt = []
    if ci + 1 < NCH:
      n1 = CHS[ci + 1]
      off = ci * KBB + KBB
      nb = (ci + 1) % 3
      nxt.append(pltpu.async_copy(ei_hbm.at[0, pl.ds(base + off, n1)],
                                  idx_v.at[nb, 0, pl.ds(0, n1)], isem))
      nxt.append(pltpu.async_copy(ei_hbm.at[1, pl.ds(base + off, n1)],
                                  idx_v.at[nb, 1, pl.ds(0, n1)], isem))
    g = pltpu.async_copy(tbl_hbm.at[idx_v.at[b, 0, pl.ds(0, n)]],
                         msg.at[mb, pl.ds(0, n)], gsem)
    for d_ in prev:
      d_.wait()
    g.wait()
    prev = [pltpu.async_copy(msg.at[mb, pl.ds(0, n)],
                             acc.at[idx_v.at[b, 1, pl.ds(0, n)]], ssem,
                             add=True)]
    for d_ in nxt:
      d_.wait()
  for d_ in prev:
    d_.wait()
  plsc.subcore_barrier()
  pltpu.sync_copy(acc.at[pl.ds(s * RPT, RPT)],
                  out_hbm.at[c, pl.ds(s * RPT, RPT)])


_agg_kernel = functools.partial(
    pl.kernel,
    out_type=jax.ShapeDtypeStruct((NC, NP, D), jnp.float32),
    mesh=_mesh,
    scratch_types=[
        pltpu.VMEM((3, 2, KBB), jnp.int32),
        pltpu.VMEM((2, KBB, D), jnp.float32),
        pltpu.VMEM_SHARED((NP, D), jnp.float32),
        pltpu.SemaphoreType.DMA,
        pltpu.SemaphoreType.DMA,
        pltpu.SemaphoreType.DMA,
    ],
    compiler_params=_sc_params,
)(_agg_body)


def _norms(dp_ref):
  deg_out = (dp_ref[0, 0] + dp_ref[1, 0])[:N, 0:1]
  deg_in = (dp_ref[0, 1] + dp_ref[1, 1])[:N, 0:1]
  ns = jnp.where(deg_out > 0, lax.rsqrt(deg_out), 0.0)
  nd = jnp.where(deg_in > 0, lax.rsqrt(deg_in), 0.0)
  return ns, nd


def _dense1_body(x_ref, w1_ref, dp_ref, hs_ref):
  ns, _ = _norms(dp_ref)
  h = jnp.dot(x_ref[...], w1_ref[...], preferred_element_type=jnp.float32)
  hs_ref[...] = h * ns


def _dense2_body(p_ref, dp_ref, b1_ref, hs2_ref):
  ns, nd = _norms(dp_ref)
  agg = p_ref[0, :N] + p_ref[1, :N]
  out1 = jnp.maximum(agg * nd + b1_ref[...], 0.0)
  hs2_ref[...] = out1 * ns


def _head_body(q_ref, dp_ref, ids_ref, w2_ref, b2_ref, wf1_ref, bf1_ref,
               wf2_ref, bf2_ref, out_ref):
  _, nd = _norms(dp_ref)
  agg = q_ref[0, :N] + q_ref[1, :N]
  h2 = jnp.dot(agg, w2_ref[...], preferred_element_type=jnp.float32)
  h2 = jnp.maximum(h2 * nd + b2_ref[...], 0.0)
  z = (ids_ref[...] == jax.lax.broadcasted_iota(jnp.int32, (1, NG), 1)
       ).astype(jnp.float32)
  sums = lax.dot_general(z, h2, (((0,), (0,)), ((), ())),
                         preferred_element_type=jnp.float32)
  cnts = lax.dot_general(z, jnp.ones((N, 1), jnp.float32),
                         (((0,), (0,)), ((), ())),
                         preferred_element_type=jnp.float32)
  pooled = sums / jnp.maximum(cnts, 1.0)
  h = jnp.dot(pooled, wf1_ref[...],
              preferred_element_type=jnp.float32) + bf1_ref[...]
  out_ref[...] = jnp.dot(h, wf2_ref[...],
                         preferred_element_type=jnp.float32) + bf2_ref[...]


def _tc_call(body, out_shape):
  return pl.pallas_call(body, out_shape=out_shape)


def kernel(x, edge_index, graph_ids, W1, b1, W2, b2, Wf1, bf1, Wf2, bf2):
  ei = edge_index.astype(jnp.int32)

  ones_b = jnp.ones((KBB, D), jnp.float32)
  zer = jnp.zeros((RPT, D), jnp.float32)

  dp4 = _deg_kernel(ei, ones_b, zer)

  hs = _tc_call(_dense1_body,
                jax.ShapeDtypeStruct((N, D), jnp.float32))(x, W1, dp4)

  p = _agg_kernel(hs, ei, zer)
  hs2 = _tc_call(_dense2_body,
                 jax.ShapeDtypeStruct((N, D), jnp.float32))(p, dp4, b1)
  q = _agg_kernel(hs2, ei, zer)

  idsp = graph_ids.astype(jnp.int32).reshape(N, 1)
  out = _tc_call(_head_body, jax.ShapeDtypeStruct((NG, 10), jnp.float32))(
      q, dp4, idsp, W2, b2, Wf1, bf1, Wf2, bf2)
  return out
